# Initial kernel scaffold; baseline (speedup 1.0000x reference)
import functools

import jax
import jax.numpy as jnp
from jax import lax
from jax.experimental import pallas as pl
from jax.experimental.pallas import tpu as pltpu
from jax.experimental.pallas import tpu_sc as plsc

N = 10000
E = 320000
D = 128
G = 128
C = 10
NC = 2
NS = 16
NW = NC * NS
EW = E // NW
CK = 80
NCH = EW // CK
NT = N // NS
DEGW = 16
ROWB = 2500

_mesh = plsc.VectorSubcoreMesh(core_axis_name="c", subcore_axis_name="s")


def _deg_body(dst_hbm, out_hbm, ones_v, dstidx_v, acc_sh):
    cid = lax.axis_index("c")
    sid = lax.axis_index("s")
    w = sid * NC + cid
    one16 = jnp.ones((16,), jnp.float32)

    def fill(i, carry):
        ones_v[i, :] = one16
        return carry

    lax.fori_loop(0, 125, fill, 0)

    def init(j, carry):
        pltpu.sync_copy(ones_v, acc_sh.at[pl.ds(sid * NT + j * 125, 125)])
        return carry

    lax.fori_loop(0, NT // 125, init, 0)
    plsc.subcore_barrier()

    def step(i, carry):
        off = w * EW + i * CK
        pltpu.sync_copy(dst_hbm.at[pl.ds(off, CK)], dstidx_v)
        pltpu.sync_copy(ones_v.at[pl.ds(0, CK)], acc_sh.at[dstidx_v], add=True)
        return carry

    lax.fori_loop(0, NCH, step, 0)
    plsc.subcore_barrier()
    pltpu.sync_copy(acc_sh.at[pl.ds(sid * NT, NT)],
                    out_hbm.at[cid, pl.ds(sid * NT, NT)])


_deg_kernel = functools.partial(
    pl.kernel,
    mesh=_mesh,
    out_type=jax.ShapeDtypeStruct((NC, N, DEGW), jnp.float32),
    scratch_types=[
        pltpu.VMEM((125, DEGW), jnp.float32),
        pltpu.VMEM((CK,), jnp.int32),
        pltpu.VMEM_SHARED((N, DEGW), jnp.float32),
    ],
)(_deg_body)


def _agg_body(feat_hbm, src_hbm, dst_hbm, out_hbm, src_v, dst_v, rows_v,
              acc_sh, sem):
    cid = lax.axis_index("c")
    sid = lax.axis_index("s")
    w = sid * NC + cid
    pltpu.sync_copy(feat_hbm.at[pl.ds(sid * NT, NT)],
                    acc_sh.at[pl.ds(sid * NT, NT)])
    plsc.subcore_barrier()

    def step(i, carry):
        off = w * EW + i * CK
        pltpu.sync_copy(src_hbm.at[pl.ds(off, CK)], src_v)
        pltpu.sync_copy(dst_hbm.at[pl.ds(off, CK)], dst_v)
        pltpu.async_copy(feat_hbm.at[src_v], rows_v, sem).wait()
        pltpu.sync_copy(rows_v, acc_sh.at[dst_v], add=True)
        return carry

    lax.fori_loop(0, NCH, step, 0)
    plsc.subcore_barrier()
    pltpu.sync_copy(acc_sh.at[pl.ds(sid * NT, NT)],
                    out_hbm.at[cid, pl.ds(sid * NT, NT)])


_agg_kernel = functools.partial(
    pl.kernel,
    mesh=_mesh,
    out_type=jax.ShapeDtypeStruct((NC, N, D), jnp.float32),
    scratch_types=[
        pltpu.VMEM((CK,), jnp.int32),
        pltpu.VMEM((CK,), jnp.int32),
        pltpu.VMEM((CK, D), jnp.float32),
        pltpu.VMEM_SHARED((N, D), jnp.float32),
        pltpu.SemaphoreType.DMA,
    ],
)(_agg_body)


def _pre_body(x_ref, w_ref, b_ref, dp_ref, p_ref, dinv_ref):
    deg = dp_ref[0] + dp_ref[1] - 1.0
    dinv = lax.rsqrt(deg)
    dinv_ref[...] = dinv
    h = jnp.dot(x_ref[...], w_ref[...],
                preferred_element_type=jnp.float32) + b_ref[...]
    p_ref[...] = h * dinv[:, 0:1]


def _mid_body(a_ref, p_ref, dinv_ref, w_ref, b_ref, g_ref, beta_ref,
              h_ref, pn_ref):
    dinv = dinv_ref[...][:, 0:1]
    agg = (a_ref[0] + a_ref[1] - p_ref[...]) * dinv
    z = jnp.dot(agg, w_ref[...],
                preferred_element_type=jnp.float32) + b_ref[...]
    mu = jnp.mean(z, axis=-1, keepdims=True)
    zc = z - mu
    var = jnp.mean(zc * zc, axis=-1, keepdims=True)
    zn = zc * lax.rsqrt(var + 1e-5)
    h = jnp.maximum(zn * g_ref[...] + beta_ref[...], 0.0)
    h_ref[...] = h
    pn_ref[...] = h * dinv


def _post_body(a_ref, p_ref, dinv_ref, w_ref, b_ref, g_ref, beta_ref,
               h1_ref, batch_ref, wpost_ref, bpost_ref, out_ref, acc_ref):
    i = pl.program_id(0)
    dinv = dinv_ref[...][:, 0:1]
    agg = (a_ref[0] + a_ref[1] - p_ref[...]) * dinv
    z = jnp.dot(agg, w_ref[...],
                preferred_element_type=jnp.float32) + b_ref[...]
    mu = jnp.mean(z, axis=-1, keepdims=True)
    zc = z - mu
    var = jnp.mean(zc * zc, axis=-1, keepdims=True)
    zn = zc * lax.rsqrt(var + 1e-5)
    h2 = jnp.maximum(zn * g_ref[...] + beta_ref[...], 0.0)
    skip = h1_ref[...] + h2
    seg = lax.broadcasted_iota(jnp.int32, (ROWB, G), 1)
    oh = jnp.where(batch_ref[...] == seg, 1.0, 0.0)
    contrib = lax.dot_general(oh, skip, (((0,), (0,)), ((), ())),
                              preferred_element_type=jnp.float32)

    @pl.when(i == 0)
    def _():
        acc_ref[...] = contrib

    @pl.when(i > 0)
    def _():
        acc_ref[...] = acc_ref[...] + contrib

    @pl.when(i == pl.num_programs(0) - 1)
    def _():
        out_ref[...] = jnp.dot(acc_ref[...], wpost_ref[...],
                               preferred_element_type=jnp.float32) + bpost_ref[...]


def _row_spec(width):
    return pl.BlockSpec((ROWB, width), lambda i: (i, 0))


def _fixed_spec(shape):
    return pl.BlockSpec(shape, lambda i: tuple(0 for _ in shape))


_PART_SPEC = pl.BlockSpec((NC, ROWB, D), lambda i: (0, i, 0))


def kernel(x, edge_index, batch, W_pre, b_pre, W1, b1, g1, beta1,
           W2, b2, g2, beta2, W_post, b_post):
    src = edge_index[0]
    dst = edge_index[1]
    grid = (N // ROWB,)

    dp = _deg_kernel(dst)

    p1, dinv16 = pl.pallas_call(
        _pre_body,
        grid=grid,
        in_specs=[
            _row_spec(D),
            _fixed_spec((D, D)),
            _fixed_spec((1, D)),
            pl.BlockSpec((NC, ROWB, DEGW), lambda i: (0, i, 0)),
        ],
        out_specs=[_row_spec(D), _row_spec(DEGW)],
        out_shape=[
            jax.ShapeDtypeStruct((N, D), jnp.float32),
            jax.ShapeDtypeStruct((N, DEGW), jnp.float32),
        ],
    )(x, W_pre, b_pre.reshape(1, D), dp)

    agg1 = _agg_kernel(p1, src, dst)

    h1, p2 = pl.pallas_call(
        _mid_body,
        grid=grid,
        in_specs=[
            _PART_SPEC,
            _row_spec(D),
            _row_spec(DEGW),
            _fixed_spec((D, D)),
            _fixed_spec((1, D)),
            _fixed_spec((1, D)),
            _fixed_spec((1, D)),
        ],
        out_specs=[_row_spec(D), _row_spec(D)],
        out_shape=[
            jax.ShapeDtypeStruct((N, D), jnp.float32),
            jax.ShapeDtypeStruct((N, D), jnp.float32),
        ],
    )(agg1, p1, dinv16, W1, b1.reshape(1, D), g1.reshape(1, D),
      beta1.reshape(1, D))

    agg2 = _agg_kernel(p2, src, dst)

    out = pl.pallas_call(
        _post_body,
        grid=grid,
        in_specs=[
            _PART_SPEC,
            _row_spec(D),
            _row_spec(DEGW),
            _fixed_spec((D, D)),
            _fixed_spec((1, D)),
            _fixed_spec((1, D)),
            _fixed_spec((1, D)),
            _row_spec(D),
            _row_spec(1),
            _fixed_spec((D, C)),
            _fixed_spec((1, C)),
        ],
        out_specs=pl.BlockSpec((G, C), lambda i: (0, 0)),
        out_shape=jax.ShapeDtypeStruct((G, C), jnp.float32),
        scratch_shapes=[pltpu.VMEM((G, G), jnp.float32)],
    )(agg2, p2, dinv16, W2, b2.reshape(1, D), g2.reshape(1, D),
      beta2.reshape(1, D), h1, batch.reshape(N, 1), W_post,
      b_post.reshape(1, C))

    return out


# speedup vs baseline: 13.1015x; 13.1015x over previous
import functools

import jax
import jax.numpy as jnp
from jax import lax
from jax.experimental import pallas as pl
from jax.experimental.pallas import tpu as pltpu
from jax.experimental.pallas import tpu_sc as plsc

N = 10000
NP = 10240
E = 320000
D = 128
G = 128
C = 10
NC = 2
NS = 16
NW = NC * NS
EW = E // NW
CK = 80
NCH = EW // CK
NT = NP // NS
DEGW = 16
ROWB = 2048

_mesh = plsc.VectorSubcoreMesh(core_axis_name="c", subcore_axis_name="s")


def _deg_body(dst_hbm, out_hbm, ones_v, dstidx_v, acc_sh):
    cid = lax.axis_index("c")
    sid = lax.axis_index("s")
    w = sid * NC + cid
    one16 = jnp.ones((16,), jnp.float32)

    def fill(i, carry):
        ones_v[i, :] = one16
        return carry

    lax.fori_loop(0, 128, fill, 0)

    def init(j, carry):
        pltpu.sync_copy(ones_v, acc_sh.at[pl.ds(sid * NT + j * 128, 128)])
        return carry

    lax.fori_loop(0, NT // 128, init, 0)
    plsc.subcore_barrier()

    def step(i, carry):
        off = w * EW + i * CK
        pltpu.sync_copy(dst_hbm.at[pl.ds(off, CK)], dstidx_v)
        pltpu.sync_copy(ones_v.at[pl.ds(0, CK)], acc_sh.at[dstidx_v], add=True)
        return carry

    lax.fori_loop(0, NCH, step, 0)
    plsc.subcore_barrier()
    pltpu.sync_copy(acc_sh.at[pl.ds(sid * NT, NT)],
                    out_hbm.at[cid, pl.ds(sid * NT, NT)])


_deg_kernel = functools.partial(
    pl.kernel,
    mesh=_mesh,
    out_type=jax.ShapeDtypeStruct((NC, NP, DEGW), jnp.float32),
    scratch_types=[
        pltpu.VMEM((128, DEGW), jnp.float32),
        pltpu.VMEM((CK,), jnp.int32),
        pltpu.VMEM_SHARED((NP, DEGW), jnp.float32),
    ],
)(_deg_body)


def _agg_body(feat_hbm, src_hbm, dst_hbm, out_hbm, src_v, dst_v, rows_v,
              acc_sh, sem):
    cid = lax.axis_index("c")
    sid = lax.axis_index("s")
    w = sid * NC + cid
    pltpu.sync_copy(feat_hbm.at[pl.ds(sid * NT, NT)],
                    acc_sh.at[pl.ds(sid * NT, NT)])
    plsc.subcore_barrier()

    def step(i, carry):
        off = w * EW + i * CK
        pltpu.sync_copy(src_hbm.at[pl.ds(off, CK)], src_v)
        pltpu.sync_copy(dst_hbm.at[pl.ds(off, CK)], dst_v)
        pltpu.async_copy(feat_hbm.at[src_v], rows_v, sem).wait()
        pltpu.sync_copy(rows_v, acc_sh.at[dst_v], add=True)
        return carry

    lax.fori_loop(0, NCH, step, 0)
    plsc.subcore_barrier()
    pltpu.sync_copy(acc_sh.at[pl.ds(sid * NT, NT)],
                    out_hbm.at[cid, pl.ds(sid * NT, NT)])


_agg_kernel = functools.partial(
    pl.kernel,
    mesh=_mesh,
    out_type=jax.ShapeDtypeStruct((NC, NP, D), jnp.float32),
    scratch_types=[
        pltpu.VMEM((CK,), jnp.int32),
        pltpu.VMEM((CK,), jnp.int32),
        pltpu.VMEM((CK, D), jnp.float32),
        pltpu.VMEM_SHARED((NP, D), jnp.float32),
        pltpu.SemaphoreType.DMA,
    ],
)(_agg_body)


def _pre_body(x_ref, w_ref, b_ref, dp_ref, p_ref, dinv_ref):
    deg = dp_ref[0] + dp_ref[1] - 1.0
    dinv = lax.rsqrt(deg)
    dinv_ref[...] = dinv
    h = jnp.dot(x_ref[...], w_ref[...],
                preferred_element_type=jnp.float32) + b_ref[...]
    p_ref[...] = h * dinv[:, 0:1]


def _mid_body(a_ref, p_ref, dinv_ref, w_ref, b_ref, g_ref, beta_ref,
              h_ref, pn_ref):
    dinv = dinv_ref[...][:, 0:1]
    agg = (a_ref[0] + a_ref[1] - p_ref[...]) * dinv
    z = jnp.dot(agg, w_ref[...],
                preferred_element_type=jnp.float32) + b_ref[...]
    mu = jnp.mean(z, axis=-1, keepdims=True)
    zc = z - mu
    var = jnp.mean(zc * zc, axis=-1, keepdims=True)
    zn = zc * lax.rsqrt(var + 1e-5)
    h = jnp.maximum(zn * g_ref[...] + beta_ref[...], 0.0)
    h_ref[...] = h
    pn_ref[...] = h * dinv


def _post_body(a_ref, p_ref, dinv_ref, w_ref, b_ref, g_ref, beta_ref,
               h1_ref, batch_ref, wpost_ref, bpost_ref, out_ref, acc_ref):
    i = pl.program_id(0)
    dinv = dinv_ref[...][:, 0:1]
    agg = (a_ref[0] + a_ref[1] - p_ref[...]) * dinv
    z = jnp.dot(agg, w_ref[...],
                preferred_element_type=jnp.float32) + b_ref[...]
    mu = jnp.mean(z, axis=-1, keepdims=True)
    zc = z - mu
    var = jnp.mean(zc * zc, axis=-1, keepdims=True)
    zn = zc * lax.rsqrt(var + 1e-5)
    h2 = jnp.maximum(zn * g_ref[...] + beta_ref[...], 0.0)
    skip = h1_ref[...] + h2
    seg = lax.broadcasted_iota(jnp.int32, (ROWB, G), 1)
    oh = jnp.where(batch_ref[...] == seg, 1.0, 0.0)
    contrib = lax.dot_general(oh, skip, (((0,), (0,)), ((), ())),
                              preferred_element_type=jnp.float32)

    @pl.when(i == 0)
    def _():
        acc_ref[...] = contrib

    @pl.when(i > 0)
    def _():
        acc_ref[...] = acc_ref[...] + contrib

    @pl.when(i == pl.num_programs(0) - 1)
    def _():
        out_ref[...] = jnp.dot(acc_ref[...], wpost_ref[...],
                               preferred_element_type=jnp.float32) + bpost_ref[...]


def _row_spec(width):
    return pl.BlockSpec((ROWB, width), lambda i: (i, 0))


def _fixed_spec(shape):
    return pl.BlockSpec(shape, lambda i: tuple(0 for _ in shape))


_PART_SPEC = pl.BlockSpec((NC, ROWB, D), lambda i: (0, i, 0))


def kernel(x, edge_index, batch, W_pre, b_pre, W1, b1, g1, beta1,
           W2, b2, g2, beta2, W_post, b_post):
    src = edge_index[0]
    dst = edge_index[1]
    x = jnp.pad(x, ((0, NP - N), (0, 0)))
    batch = jnp.pad(batch, (0, NP - N), constant_values=G)
    grid = (NP // ROWB,)

    dp = _deg_kernel(dst)

    p1, dinv16 = pl.pallas_call(
        _pre_body,
        grid=grid,
        in_specs=[
            _row_spec(D),
            _fixed_spec((D, D)),
            _fixed_spec((1, D)),
            pl.BlockSpec((NC, ROWB, DEGW), lambda i: (0, i, 0)),
        ],
        out_specs=[_row_spec(D), _row_spec(DEGW)],
        out_shape=[
            jax.ShapeDtypeStruct((NP, D), jnp.float32),
            jax.ShapeDtypeStruct((NP, DEGW), jnp.float32),
        ],
    )(x, W_pre, b_pre.reshape(1, D), dp)

    agg1 = _agg_kernel(p1, src, dst)

    h1, p2 = pl.pallas_call(
        _mid_body,
        grid=grid,
        in_specs=[
            _PART_SPEC,
            _row_spec(D),
            _row_spec(DEGW),
            _fixed_spec((D, D)),
            _fixed_spec((1, D)),
            _fixed_spec((1, D)),
            _fixed_spec((1, D)),
        ],
        out_specs=[_row_spec(D), _row_spec(D)],
        out_shape=[
            jax.ShapeDtypeStruct((NP, D), jnp.float32),
            jax.ShapeDtypeStruct((NP, D), jnp.float32),
        ],
    )(agg1, p1, dinv16, W1, b1.reshape(1, D), g1.reshape(1, D),
      beta1.reshape(1, D))

    agg2 = _agg_kernel(p2, src, dst)

    out = pl.pallas_call(
        _post_body,
        grid=grid,
        in_specs=[
            _PART_SPEC,
            _row_spec(D),
            _row_spec(DEGW),
            _fixed_spec((D, D)),
            _fixed_spec((1, D)),
            _fixed_spec((1, D)),
            _fixed_spec((1, D)),
            _row_spec(D),
            _row_spec(1),
            _fixed_spec((D, C)),
            _fixed_spec((1, C)),
        ],
        out_specs=pl.BlockSpec((G, C), lambda i: (0, 0)),
        out_shape=jax.ShapeDtypeStruct((G, C), jnp.float32),
        scratch_shapes=[pltpu.VMEM((G, G), jnp.float32)],
    )(agg2, p2, dinv16, W2, b2.reshape(1, D), g2.reshape(1, D),
      beta2.reshape(1, D), h1, batch.reshape(NP, 1), W_post,
      b_post.reshape(1, C))

    return out


# speedup vs baseline: 23.8296x; 1.8188x over previous
import functools

import jax
import jax.numpy as jnp
from jax import lax
from jax.experimental import pallas as pl
from jax.experimental.pallas import tpu as pltpu
from jax.experimental.pallas import tpu_sc as plsc

N = 10000
NP = 10240
E = 320000
D = 128
G = 128
C = 10
NC = 2
NS = 16
NW = NC * NS
EW = E // NW
CK = 64
NCHP = -(-EW // CK)
EWP = NCHP * CK
NT = NP // NS
DEGW = 16
ROWB = 2048

_mesh = plsc.VectorSubcoreMesh(core_axis_name="c", subcore_axis_name="s")


def _deg_body(dst_hbm, out_hbm, ones_v, didx_v, acc_sh, dsem0, dsem1):
    cid = lax.axis_index("c")
    sid = lax.axis_index("s")
    w = sid * NC + cid
    one16 = jnp.ones((16,), jnp.float32)

    def fill(i, carry):
        ones_v[i, :] = one16
        return carry

    lax.fori_loop(0, 128, fill, 0)

    def init(j, carry):
        pltpu.sync_copy(ones_v, acc_sh.at[pl.ds(sid * NT + j * 128, 128)])
        return carry

    lax.fori_loop(0, NT // 128, init, 0)
    pltpu.sync_copy(dst_hbm.at[w], didx_v)
    plsc.subcore_barrier()

    dsems = (dsem0, dsem1)

    def dscat_start(i, b):
        pltpu.async_copy(ones_v.at[pl.ds(0, CK)], acc_sh.at[didx_v.at[i]],
                         dsems[b], add=True)

    def dscat_wait(i, b):
        pltpu.make_async_copy(ones_v.at[pl.ds(0, CK)],
                              acc_sh.at[didx_v.at[i]], dsems[b]).wait()

    dscat_start(0, 0)
    dscat_start(1, 1)

    def pair(j, carry):
        for k in range(2):
            i = 2 * j + k + 2

            @pl.when(i < NCHP)
            def _(i=i, k=k):
                dscat_wait(i, k)
                dscat_start(i, k)
        return carry

    lax.fori_loop(0, (NCHP - 1) // 2, pair, 0)
    dscat_wait(0, 0)
    dscat_wait(0, 1)
    plsc.subcore_barrier()
    pltpu.sync_copy(acc_sh.at[pl.ds(sid * NT, NT)],
                    out_hbm.at[cid, pl.ds(sid * NT, NT)])


_deg_kernel = functools.partial(
    pl.kernel,
    mesh=_mesh,
    out_type=jax.ShapeDtypeStruct((NC, NP, DEGW), jnp.float32),
    scratch_types=[
        pltpu.VMEM((128, DEGW), jnp.float32),
        pltpu.VMEM((NCHP, CK), jnp.int32),
        pltpu.VMEM_SHARED((NP, DEGW), jnp.float32),
        pltpu.SemaphoreType.DMA,
        pltpu.SemaphoreType.DMA,
    ],
)(_deg_body)


def _agg_body(feat_hbm, ei_hbm, out_hbm, ei_v, rows_v, acc_sh,
              isem0, isem1, isem2, isem3,
              gsem0, gsem1, gsem2, gsem3,
              ssem0, ssem1, ssem2, ssem3):
    cid = lax.axis_index("c")
    sid = lax.axis_index("s")
    w = sid * NC + cid
    isems = (isem0, isem1, isem2, isem3)
    gsems = (gsem0, gsem1, gsem2, gsem3)
    ssems = (ssem0, ssem1, ssem2, ssem3)

    def idx_start(i, c):
        pltpu.async_copy(ei_hbm.at[w, i], ei_v.at[c], isems[c])

    def idx_wait(i, c):
        pltpu.make_async_copy(ei_hbm.at[w, i], ei_v.at[c], isems[c]).wait()

    def gather_start(c):
        pltpu.async_copy(feat_hbm.at[ei_v.at[c, 0]], rows_v.at[c], gsems[c])

    def gather_wait(c):
        pltpu.make_async_copy(feat_hbm.at[ei_v.at[c, 0]], rows_v.at[c],
                              gsems[c]).wait()

    def scat_start(c):
        pltpu.async_copy(rows_v.at[c], acc_sh.at[ei_v.at[c, 1]], ssems[c],
                         add=True)

    def scat_wait(c):
        pltpu.make_async_copy(rows_v.at[c], acc_sh.at[ei_v.at[c, 1]],
                              ssems[c]).wait()

    pltpu.sync_copy(feat_hbm.at[pl.ds(sid * NT, NT)],
                    acc_sh.at[pl.ds(sid * NT, NT)])
    plsc.subcore_barrier()

    idx_start(0, 0)
    idx_start(1, 1)
    idx_wait(0, 0)
    gather_start(0)

    def quad(q, carry):
        for k in range(4):
            r = 4 * q + k
            c0 = k
            c1 = (k + 1) % 4
            c2 = (k + 2) % 4

            @pl.when(r + 2 < NCHP)
            def _(r=r, c2=c2):
                @pl.when(r >= 2)
                def _():
                    scat_wait(c2)

                idx_start(r + 2, c2)

            @pl.when(r + 1 < NCHP)
            def _(r=r, c1=c1):
                idx_wait(r + 1, c1)
                gather_start(c1)

            @pl.when(r < NCHP)
            def _(r=r, c0=c0):
                gather_wait(c0)
                scat_start(c0)
        return carry

    lax.fori_loop(0, (NCHP + 3) // 4, quad, 0)
    for t in range(4):
        scat_wait((NCHP - 4 + t) % 4)
    plsc.subcore_barrier()
    pltpu.sync_copy(acc_sh.at[pl.ds(sid * NT, NT)],
                    out_hbm.at[cid, pl.ds(sid * NT, NT)])


_agg_kernel = functools.partial(
    pl.kernel,
    mesh=_mesh,
    out_type=jax.ShapeDtypeStruct((NC, NP, D), jnp.float32),
    scratch_types=[
        pltpu.VMEM((4, 2, CK), jnp.int32),
        pltpu.VMEM((4, CK, D), jnp.float32),
        pltpu.VMEM_SHARED((NP, D), jnp.float32),
    ] + [pltpu.SemaphoreType.DMA] * 12,
)(_agg_body)


def _pre_body(x_ref, w_ref, b_ref, dp_ref, p_ref, dinv_ref):
    deg = dp_ref[0] + dp_ref[1] - 1.0
    rowid = (pl.program_id(0) * ROWB
             + lax.broadcasted_iota(jnp.int32, (ROWB, DEGW), 0))
    dinv = jnp.where(rowid < N, lax.rsqrt(deg), 0.0)
    dinv_ref[...] = dinv
    h = jnp.dot(x_ref[...], w_ref[...],
                preferred_element_type=jnp.float32) + b_ref[...]
    p_ref[...] = h * dinv[:, 0:1]


def _mid_body(a_ref, p_ref, dinv_ref, w_ref, b_ref, g_ref, beta_ref,
              h_ref, pn_ref):
    dinv = dinv_ref[...][:, 0:1]
    agg = (a_ref[0] + a_ref[1] - p_ref[...]) * dinv
    z = jnp.dot(agg, w_ref[...],
                preferred_element_type=jnp.float32) + b_ref[...]
    mu = jnp.mean(z, axis=-1, keepdims=True)
    zc = z - mu
    var = jnp.mean(zc * zc, axis=-1, keepdims=True)
    zn = zc * lax.rsqrt(var + 1e-5)
    h = jnp.maximum(zn * g_ref[...] + beta_ref[...], 0.0)
    h_ref[...] = h
    pn_ref[...] = h * dinv


def _post_body(a_ref, p_ref, dinv_ref, w_ref, b_ref, g_ref, beta_ref,
               h1_ref, batch_ref, wpost_ref, bpost_ref, out_ref, acc_ref):
    i = pl.program_id(0)
    dinv = dinv_ref[...][:, 0:1]
    agg = (a_ref[0] + a_ref[1] - p_ref[...]) * dinv
    z = jnp.dot(agg, w_ref[...],
                preferred_element_type=jnp.float32) + b_ref[...]
    mu = jnp.mean(z, axis=-1, keepdims=True)
    zc = z - mu
    var = jnp.mean(zc * zc, axis=-1, keepdims=True)
    zn = zc * lax.rsqrt(var + 1e-5)
    h2 = jnp.maximum(zn * g_ref[...] + beta_ref[...], 0.0)
    skip = h1_ref[...] + h2
    seg = lax.broadcasted_iota(jnp.int32, (ROWB, G), 1)
    oh = jnp.where(batch_ref[...] == seg, 1.0, 0.0)
    contrib = lax.dot_general(oh, skip, (((0,), (0,)), ((), ())),
                              preferred_element_type=jnp.float32)

    @pl.when(i == 0)
    def _():
        acc_ref[...] = contrib

    @pl.when(i > 0)
    def _():
        acc_ref[...] = acc_ref[...] + contrib

    @pl.when(i == pl.num_programs(0) - 1)
    def _():
        out_ref[...] = jnp.dot(acc_ref[...], wpost_ref[...],
                               preferred_element_type=jnp.float32) + bpost_ref[...]


def _row_spec(width):
    return pl.BlockSpec((ROWB, width), lambda i: (i, 0))


def _fixed_spec(shape):
    return pl.BlockSpec(shape, lambda i: tuple(0 for _ in shape))


_PART_SPEC = pl.BlockSpec((NC, ROWB, D), lambda i: (0, i, 0))


def kernel(x, edge_index, batch, W_pre, b_pre, W1, b1, g1, beta1,
           W2, b2, g2, beta2, W_post, b_post):
    srcw = edge_index[0].reshape(NW, EW)
    dstw = edge_index[1].reshape(NW, EW)
    srcp = jnp.pad(srcw, ((0, 0), (0, EWP - EW)), constant_values=N)
    dstp = jnp.pad(dstw, ((0, 0), (0, EWP - EW)), constant_values=N)
    ei = jnp.stack([srcp.reshape(NW, NCHP, CK),
                    dstp.reshape(NW, NCHP, CK)], axis=2)
    dst_blocks = dstp.reshape(NW, NCHP, CK)
    x = jnp.pad(x, ((0, NP - N), (0, 0)))
    batch = jnp.pad(batch, (0, NP - N), constant_values=G)
    grid = (NP // ROWB,)

    dp = _deg_kernel(dst_blocks)

    p1, dinv16 = pl.pallas_call(
        _pre_body,
        grid=grid,
        in_specs=[
            _row_spec(D),
            _fixed_spec((D, D)),
            _fixed_spec((1, D)),
            pl.BlockSpec((NC, ROWB, DEGW), lambda i: (0, i, 0)),
        ],
        out_specs=[_row_spec(D), _row_spec(DEGW)],
        out_shape=[
            jax.ShapeDtypeStruct((NP, D), jnp.float32),
            jax.ShapeDtypeStruct((NP, DEGW), jnp.float32),
        ],
    )(x, W_pre, b_pre.reshape(1, D), dp)

    agg1 = _agg_kernel(p1, ei)

    h1, p2 = pl.pallas_call(
        _mid_body,
        grid=grid,
        in_specs=[
            _PART_SPEC,
            _row_spec(D),
            _row_spec(DEGW),
            _fixed_spec((D, D)),
            _fixed_spec((1, D)),
            _fixed_spec((1, D)),
            _fixed_spec((1, D)),
        ],
        out_specs=[_row_spec(D), _row_spec(D)],
        out_shape=[
            jax.ShapeDtypeStruct((NP, D), jnp.float32),
            jax.ShapeDtypeStruct((NP, D), jnp.float32),
        ],
    )(agg1, p1, dinv16, W1, b1.reshape(1, D), g1.reshape(1, D),
      beta1.reshape(1, D))

    agg2 = _agg_kernel(p2, ei)

    out = pl.pallas_call(
        _post_body,
        grid=grid,
        in_specs=[
            _PART_SPEC,
            _row_spec(D),
            _row_spec(DEGW),
            _fixed_spec((D, D)),
            _fixed_spec((1, D)),
            _fixed_spec((1, D)),
            _fixed_spec((1, D)),
            _row_spec(D),
            _row_spec(1),
            _fixed_spec((D, C)),
            _fixed_spec((1, C)),
        ],
        out_specs=pl.BlockSpec((G, C), lambda i: (0, 0)),
        out_shape=jax.ShapeDtypeStruct((G, C), jnp.float32),
        scratch_shapes=[pltpu.VMEM((G, G), jnp.float32)],
    )(agg2, p2, dinv16, W2, b2.reshape(1, D), g2.reshape(1, D),
      beta2.reshape(1, D), h1, batch.reshape(NP, 1), W_post,
      b_post.reshape(1, C))

    return out
